# Initial kernel scaffold; baseline (speedup 1.0000x reference)
import jax
import jax.numpy as jnp
from jax.experimental import pallas as pl


def kernel(z, edges):
    raise NotImplementedError("write your pallas kernel here")


# speedup vs baseline: 1.2372x; 1.2372x over previous
import functools

import jax
import jax.numpy as jnp
from jax import lax
from jax.experimental import pallas as pl
from jax.experimental.pallas import tpu as pltpu
from jax.experimental.pallas import tpu_sc as plsc

N_NODES = 10000
D = 256
E = 160000

NC = 2
NS = 16
NW = NC * NS
L = 16
DPW = D // NW

CH = 2000
NCHUNK = E // CH

_mesh = plsc.VectorSubcoreMesh(core_axis_name="c", subcore_axis_name="s")


@functools.partial(
    pl.kernel,
    mesh=_mesh,
    compiler_params=pltpu.CompilerParams(
        needs_layout_passes=False, use_tc_tiling_on_sc=False
    ),
    out_type=jax.ShapeDtypeStruct((NW * E,), jnp.float32),
    scratch_types=[
        pltpu.VMEM((N_NODES * DPW,), jnp.float32),
        pltpu.VMEM((CH,), jnp.int32),
        pltpu.VMEM((CH,), jnp.int32),
        pltpu.VMEM((CH,), jnp.float32),
    ],
)
def _sc_partial_dots(zt_hbm, u_hbm, v_hbm, out_hbm, zt, ub, vb, pb):
    cid = lax.axis_index("c")
    sid = lax.axis_index("s")
    wid = sid * NC + cid

    pltpu.sync_copy(zt_hbm.at[pl.ds(wid * (N_NODES * DPW), N_NODES * DPW)], zt)

    cols = [jnp.full((L,), d, jnp.int32) for d in range(DPW)]
    zero = jnp.zeros((L,), jnp.float32)

    def chunk_body(c, carry):
        base = c * CH
        pltpu.sync_copy(u_hbm.at[pl.ds(base, CH)], ub)
        pltpu.sync_copy(v_hbm.at[pl.ds(base, CH)], vb)

        def j_body(j, inner):
            off = j * L
            u16 = ub[pl.ds(off, L)] * DPW
            v16 = vb[pl.ds(off, L)] * DPW
            acc = zero
            for d in range(DPW):
                a = plsc.load_gather(zt, [u16 + cols[d]])
                b = plsc.load_gather(zt, [v16 + cols[d]])
                acc = acc + a * b
            pb[pl.ds(off, L)] = acc
            return inner

        lax.fori_loop(0, CH // L, j_body, 0)
        pltpu.sync_copy(pb, out_hbm.at[pl.ds(wid * E + base, CH)])
        return carry

    lax.fori_loop(0, NCHUNK, chunk_body, 0)


def _tc_sum_kernel(parts_ref, o_ref):
    o_ref[...] = jnp.sum(parts_ref[...], axis=0, keepdims=True)


_BLK = 16000


def _tc_sum(parts):
    out = pl.pallas_call(
        _tc_sum_kernel,
        grid=(E // _BLK,),
        in_specs=[pl.BlockSpec((NW, _BLK), lambda i: (0, i))],
        out_specs=pl.BlockSpec((1, _BLK), lambda i: (0, i)),
        out_shape=jax.ShapeDtypeStruct((1, E), jnp.float32),
    )(parts)
    return out[0]


def kernel(z, edges):
    edges = edges.astype(jnp.int32)
    u = edges[:, 0]
    v = edges[:, 1]
    zt = jnp.transpose(z.reshape(N_NODES, NW, DPW), (1, 0, 2)).reshape(-1)
    parts = _sc_partial_dots(zt, u, v)
    return _tc_sum(parts.reshape(NW, E))


# speedup vs baseline: 1.4818x; 1.1977x over previous
import functools

import jax
import jax.numpy as jnp
from jax import lax
from jax.experimental import pallas as pl
from jax.experimental.pallas import tpu as pltpu
from jax.experimental.pallas import tpu_sc as plsc

N_NODES = 10000
D = 256
E = 160000

NC = 2
NS = 16
NW = NC * NS
L = 16
DPW = D // NW

CH = 4000
NCHUNK = E // CH

_mesh = plsc.VectorSubcoreMesh(core_axis_name="c", subcore_axis_name="s")


@functools.partial(
    pl.kernel,
    mesh=_mesh,
    compiler_params=pltpu.CompilerParams(
        needs_layout_passes=False, use_tc_tiling_on_sc=False
    ),
    out_type=jax.ShapeDtypeStruct((NW * E,), jnp.float32),
    scratch_types=[
        pltpu.VMEM((N_NODES * DPW,), jnp.float32),
        pltpu.VMEM((CH,), jnp.int32),
        pltpu.VMEM((CH,), jnp.int32),
        pltpu.VMEM((CH,), jnp.int32),
        pltpu.VMEM((CH,), jnp.int32),
        pltpu.VMEM((CH,), jnp.float32),
        pltpu.VMEM((CH,), jnp.float32),
        pltpu.SemaphoreType.DMA,
        pltpu.SemaphoreType.DMA,
        pltpu.SemaphoreType.DMA,
        pltpu.SemaphoreType.DMA,
    ],
)
def _sc_partial_dots(
    zt_hbm, u_hbm, v_hbm, out_hbm,
    zt, ub0, ub1, vb0, vb1, pb0, pb1,
    in_sem0, in_sem1, out_sem0, out_sem1,
):
    cid = lax.axis_index("c")
    sid = lax.axis_index("s")
    wid = sid * NC + cid

    ubs = (ub0, ub1)
    vbs = (vb0, vb1)
    pbs = (pb0, pb1)
    in_sems = (in_sem0, in_sem1)
    out_sems = (out_sem0, out_sem1)

    pltpu.sync_copy(zt_hbm.at[pl.ds(wid * (N_NODES * DPW), N_NODES * DPW)], zt)

    def start_in(c, b):
        pltpu.async_copy(u_hbm.at[pl.ds(c * CH, CH)], ubs[b], in_sems[b])
        pltpu.async_copy(v_hbm.at[pl.ds(c * CH, CH)], vbs[b], in_sems[b])

    def wait_in(b):
        pltpu.make_async_copy(u_hbm.at[pl.ds(0, CH)], ubs[b], in_sems[b]).wait()
        pltpu.make_async_copy(v_hbm.at[pl.ds(0, CH)], vbs[b], in_sems[b]).wait()

    def start_out(c, b):
        pltpu.async_copy(
            pbs[b], out_hbm.at[pl.ds(wid * E + c * CH, CH)], out_sems[b]
        )

    def wait_out(b):
        pltpu.make_async_copy(
            pbs[b], out_hbm.at[pl.ds(0, CH)], out_sems[b]
        ).wait()

    cols = [jnp.full((L,), d, jnp.int32) for d in range(DPW)]
    zero = jnp.zeros((L,), jnp.float32)

    def compute_chunk(b):
        ub = ubs[b]
        vb = vbs[b]
        pb = pbs[b]

        def j_body(j, inner):
            off = j * L
            u16 = ub[pl.ds(off, L)] * DPW
            v16 = vb[pl.ds(off, L)] * DPW
            acc = zero
            for d in range(DPW):
                a = plsc.load_gather(zt, [u16 + cols[d]])
                bv = plsc.load_gather(zt, [v16 + cols[d]])
                acc = acc + a * bv
            pb[pl.ds(off, L)] = acc
            return inner

        lax.fori_loop(0, CH // L, j_body, 0)

    start_in(0, 0)

    def chunk_body(g):
        for b in range(2):
            c = g + b

            @pl.when(c + 1 < NCHUNK)
            def _():
                start_in(c + 1, 1 - b)

            wait_in(b)

            @pl.when(c >= 2)
            def _():
                wait_out(b)

            compute_chunk(b)
            start_out(c, b)

    pl.loop(0, NCHUNK, step=2)(chunk_body)

    wait_out(0)
    wait_out(1)


def _tc_sum_kernel(parts_ref, o_ref):
    o_ref[...] = jnp.sum(parts_ref[...], axis=0, keepdims=True)


_BLK = 16000


def _tc_sum(parts):
    out = pl.pallas_call(
        _tc_sum_kernel,
        grid=(E // _BLK,),
        in_specs=[pl.BlockSpec((NW, _BLK), lambda i: (0, i))],
        out_specs=pl.BlockSpec((1, _BLK), lambda i: (0, i)),
        out_shape=jax.ShapeDtypeStruct((1, E), jnp.float32),
    )(parts)
    return out[0]


def kernel(z, edges):
    edges = edges.astype(jnp.int32)
    u = edges[:, 0]
    v = edges[:, 1]
    zt = jnp.transpose(z.reshape(N_NODES, NW, DPW), (1, 0, 2)).reshape(-1)
    parts = _sc_partial_dots(zt, u, v)
    return _tc_sum(parts.reshape(NW, E))


# speedup vs baseline: 3.2636x; 2.2024x over previous
import functools

import jax
import jax.numpy as jnp
from jax import lax
from jax.experimental import pallas as pl
from jax.experimental.pallas import tpu as pltpu
from jax.experimental.pallas import tpu_sc as plsc

N_NODES = 10000
D = 256
E = 160000

NC = 2
NS = 16
NW = NC * NS
L = 16
DPW = D // NW

CH = 4000
NCHUNK = E // CH

_mesh = plsc.VectorSubcoreMesh(core_axis_name="c", subcore_axis_name="s")


@functools.partial(
    pl.kernel,
    mesh=_mesh,
    compiler_params=pltpu.CompilerParams(
        needs_layout_passes=False, use_tc_tiling_on_sc=False
    ),
    out_type=jax.ShapeDtypeStruct((NW * E,), jnp.float32),
    scratch_types=[
        pltpu.VMEM((N_NODES * DPW,), jnp.float32),
        pltpu.VMEM((CH,), jnp.int32),
        pltpu.VMEM((CH,), jnp.int32),
        pltpu.VMEM((CH,), jnp.int32),
        pltpu.VMEM((CH,), jnp.int32),
        pltpu.VMEM((CH,), jnp.float32),
        pltpu.VMEM((CH,), jnp.float32),
        pltpu.SemaphoreType.DMA,
        pltpu.SemaphoreType.DMA,
        pltpu.SemaphoreType.DMA,
        pltpu.SemaphoreType.DMA,
    ],
)
def _sc_partial_dots(
    zt_hbm, u_hbm, v_hbm, out_hbm,
    zt, ub0, ub1, vb0, vb1, pb0, pb1,
    in_sem0, in_sem1, out_sem0, out_sem1,
):
    cid = lax.axis_index("c")
    sid = lax.axis_index("s")
    wid = sid * NC + cid

    ubs = (ub0, ub1)
    vbs = (vb0, vb1)
    pbs = (pb0, pb1)
    in_sems = (in_sem0, in_sem1)
    out_sems = (out_sem0, out_sem1)

    pltpu.sync_copy(zt_hbm.at[pl.ds(wid * (N_NODES * DPW), N_NODES * DPW)], zt)

    def start_in(c, b):
        pltpu.async_copy(u_hbm.at[pl.ds(c * CH, CH)], ubs[b], in_sems[b])
        pltpu.async_copy(v_hbm.at[pl.ds(c * CH, CH)], vbs[b], in_sems[b])

    def wait_in(b):
        pltpu.make_async_copy(u_hbm.at[pl.ds(0, CH)], ubs[b], in_sems[b]).wait()
        pltpu.make_async_copy(v_hbm.at[pl.ds(0, CH)], vbs[b], in_sems[b]).wait()

    def start_out(c, b):
        pltpu.async_copy(
            pbs[b], out_hbm.at[pl.ds(wid * E + c * CH, CH)], out_sems[b]
        )

    def wait_out(b):
        pltpu.make_async_copy(
            pbs[b], out_hbm.at[pl.ds(0, CH)], out_sems[b]
        ).wait()

    cols = [jnp.full((L,), d * N_NODES, jnp.int32) for d in range(DPW)]
    zero = jnp.zeros((L,), jnp.float32)

    def compute_chunk(b):
        ub = ubs[b]
        vb = vbs[b]
        pb = pbs[b]

        def j_body(j, inner):
            off = j * L
            u16 = ub[pl.ds(off, L)]
            v16 = vb[pl.ds(off, L)]
            acc = zero
            for d in range(DPW):
                a = plsc.load_gather(zt, [u16 + cols[d]])
                bv = plsc.load_gather(zt, [v16 + cols[d]])
                acc = acc + a * bv
            pb[pl.ds(off, L)] = acc
            return inner

        lax.fori_loop(0, CH // L, j_body, 0)

    start_in(0, 0)

    def chunk_body(g):
        for b in range(2):
            c = g + b

            @pl.when(c + 1 < NCHUNK)
            def _():
                start_in(c + 1, 1 - b)

            wait_in(b)

            @pl.when(c >= 2)
            def _():
                wait_out(b)

            compute_chunk(b)
            start_out(c, b)

    pl.loop(0, NCHUNK, step=2)(chunk_body)

    wait_out(0)
    wait_out(1)


def _tc_sum_kernel(parts_ref, o_ref):
    o_ref[...] = jnp.sum(parts_ref[...], axis=0, keepdims=True)


_BLK = 16000


def _tc_sum(parts):
    out = pl.pallas_call(
        _tc_sum_kernel,
        grid=(E // _BLK,),
        in_specs=[pl.BlockSpec((NW, _BLK), lambda i: (0, i))],
        out_specs=pl.BlockSpec((1, _BLK), lambda i: (0, i)),
        out_shape=jax.ShapeDtypeStruct((1, E), jnp.float32),
    )(parts)
    return out[0]


def kernel(z, edges):
    edges = edges.astype(jnp.int32)
    u = edges[:, 0]
    v = edges[:, 1]
    zt = z.T.reshape(-1)
    parts = _sc_partial_dots(zt, u, v)
    return _tc_sum(parts.reshape(NW, E))
